# baseline (device time: 152489 ns/iter reference)
import jax
import jax.numpy as jnp
from jax import lax
from jax.experimental import pallas as pl
from jax.experimental.pallas import tpu as pltpu

N_DEV = 8
_PIPES = (
    dict(stages=((3, 1), (1, 0), (4, 2))),
    dict(stages=((4, 2), (3, 1), (1, 0))),
)


def kernel(table, idx):
    rows_per, d = table.shape
    n = idx.shape[0]

    my = lax.axis_index("i")
    local = idx - my * rows_per
    valid = (local >= 0) & (local < rows_per)
    safe = jnp.where(valid, local, 0)
    rows = jnp.take(table, safe, axis=0, mode="clip")
    partial = jnp.where(valid[:, None], rows, 0.0).astype(jnp.bfloat16)

    half = n // 2
    sizes = (half // 2, half // 4, half // 8)
    st_sz = sum(sizes)
    st_offs = (0, sizes[0], sizes[0] + sizes[1])

    def body(p_ref, out_ref, stage_ref, send_sems, recv_sems):
        p = lax.axis_index("i")

        barrier_sem = pltpu.get_barrier_semaphore()
        for m in (1, 3, 4):
            pl.semaphore_signal(
                barrier_sem, inc=1,
                device_id=(p ^ m,), device_id_type=pl.DeviceIdType.MESH,
            )
        pl.semaphore_wait(barrier_sem, 3)

        bits = [
            [(p >> bit) & 1 for _, bit in pipe["stages"]] for pipe in _PIPES
        ]

        def cond_for(pi, combo):
            c = bits[pi][0] == combo[0]
            for j in range(1, len(combo)):
                c = jnp.logical_and(c, bits[pi][j] == combo[j])
            return c

        def combos(k):
            out = [()]
            for _ in range(k):
                out = [c + (v,) for c in out for v in (0, 1)]
            return out

        def rs_rdma(pi, s, combo):
            base = pi * half
            keep_base = base + sum(combo[j] * sizes[j] for j in range(s))
            send_off = keep_base + (1 - combo[s]) * sizes[s]
            src = p_ref if s == 0 else out_ref
            return pltpu.make_async_remote_copy(
                src_ref=src.at[pl.ds(send_off, sizes[s]), :],
                dst_ref=stage_ref.at[pl.ds(pi * st_sz + st_offs[s], sizes[s]), :],
                send_sem=send_sems.at[pi * 6 + s],
                recv_sem=recv_sems.at[pi * 6 + s],
                device_id=(p ^ _PIPES[pi]["stages"][s][0],),
                device_id_type=pl.DeviceIdType.MESH,
            )

        def rs_add(pi, s, combo):
            keep = pi * half + sum(combo[j] * sizes[j] for j in range(s + 1))
            out_ref[pl.ds(keep, sizes[s]), :] = (
                out_ref[pl.ds(keep, sizes[s]), :]
                + stage_ref[pl.ds(pi * st_sz + st_offs[s], sizes[s]), :]
            )

        def ag_rdma(pi, s, combo, ag_i):
            off = pi * half + sum(combo[j] * sizes[j] for j in range(s + 1))
            return pltpu.make_async_remote_copy(
                src_ref=out_ref.at[pl.ds(off, sizes[s]), :],
                dst_ref=out_ref.at[pl.ds(off, sizes[s]), :],
                send_sem=send_sems.at[pi * 6 + 3 + ag_i],
                recv_sem=recv_sems.at[pi * 6 + 3 + ag_i],
                device_id=(p ^ _PIPES[pi]["stages"][s][0],),
                device_id_type=pl.DeviceIdType.MESH,
            )

        for pi in range(2):
            for combo in combos(1):
                @pl.when(cond_for(pi, combo))
                def _(pi=pi, combo=combo):
                    rs_rdma(pi, 0, combo).start()

        out_ref[...] = p_ref[...]

        for pi in range(2):
            for combo in combos(1):
                @pl.when(cond_for(pi, combo))
                def _(pi=pi, combo=combo):
                    rs_rdma(pi, 0, combo).wait()
                    rs_add(pi, 0, combo)

        for s in (1, 2):
            for pi in range(2):
                for combo in combos(s + 1):
                    @pl.when(cond_for(pi, combo))
                    def _(pi=pi, s=s, combo=combo):
                        rs_rdma(pi, s, combo).start()
            for pi in range(2):
                for combo in combos(s + 1):
                    @pl.when(cond_for(pi, combo))
                    def _(pi=pi, s=s, combo=combo):
                        rs_rdma(pi, s, combo).wait()
                        rs_add(pi, s, combo)

        for ag_i, s in enumerate((2, 1, 0)):
            for pi in range(2):
                for combo in combos(s + 1):
                    @pl.when(cond_for(pi, combo))
                    def _(pi=pi, s=s, combo=combo, ag_i=ag_i):
                        ag_rdma(pi, s, combo, ag_i).start()
            for pi in range(2):
                for combo in combos(s + 1):
                    @pl.when(cond_for(pi, combo))
                    def _(pi=pi, s=s, combo=combo, ag_i=ag_i):
                        ag_rdma(pi, s, combo, ag_i).wait()

    return pl.pallas_call(
        body,
        out_shape=jax.ShapeDtypeStruct((n, d), jnp.bfloat16),
        in_specs=[pl.BlockSpec(memory_space=pltpu.VMEM)],
        out_specs=pl.BlockSpec(memory_space=pltpu.VMEM),
        scratch_shapes=[
            pltpu.VMEM((2 * st_sz, d), jnp.bfloat16),
            pltpu.SemaphoreType.DMA((12,)),
            pltpu.SemaphoreType.DMA((12,)),
        ],
        compiler_params=pltpu.CompilerParams(collective_id=0),
    )(partial)


# device time: 77359 ns/iter; 1.9712x vs baseline; 1.9712x over previous
import jax
import jax.numpy as jnp
from jax import lax
from jax.experimental import pallas as pl
from jax.experimental.pallas import tpu as pltpu

N_DEV = 8
_PIPES = (
    dict(stages=((3, 1), (1, 0), (4, 2))),
    dict(stages=((4, 2), (3, 1), (1, 0))),
)


def kernel(table, idx):
    rows_per, d = table.shape
    n = idx.shape[0]

    my = lax.axis_index("i")
    local = idx - my * rows_per
    valid = (local >= 0) & (local < rows_per)
    safe = jnp.where(valid, local, 0)
    rows = jnp.take(table.astype(jnp.bfloat16), safe, axis=0, mode="clip")
    partial = jnp.where(valid[:, None], rows, jnp.bfloat16(0))

    half = n // 2
    sizes = (half // 2, half // 4, half // 8)
    st_sz = sum(sizes)
    st_offs = (0, sizes[0], sizes[0] + sizes[1])

    def body(p_ref, out_ref, stage_ref, send_sems, recv_sems):
        p = lax.axis_index("i")

        barrier_sem = pltpu.get_barrier_semaphore()
        for m in (1, 3, 4):
            pl.semaphore_signal(
                barrier_sem, inc=1,
                device_id=(p ^ m,), device_id_type=pl.DeviceIdType.MESH,
            )
        pl.semaphore_wait(barrier_sem, 3)

        bits = [
            [(p >> bit) & 1 for _, bit in pipe["stages"]] for pipe in _PIPES
        ]

        def cond_for(pi, combo):
            c = bits[pi][0] == combo[0]
            for j in range(1, len(combo)):
                c = jnp.logical_and(c, bits[pi][j] == combo[j])
            return c

        def combos(k):
            out = [()]
            for _ in range(k):
                out = [c + (v,) for c in out for v in (0, 1)]
            return out

        def rs_rdma(pi, s, combo):
            base = pi * half
            keep_base = base + sum(combo[j] * sizes[j] for j in range(s))
            send_off = keep_base + (1 - combo[s]) * sizes[s]
            src = p_ref if s == 0 else out_ref
            return pltpu.make_async_remote_copy(
                src_ref=src.at[pl.ds(send_off, sizes[s]), :],
                dst_ref=stage_ref.at[pl.ds(pi * st_sz + st_offs[s], sizes[s]), :],
                send_sem=send_sems.at[pi * 6 + s],
                recv_sem=recv_sems.at[pi * 6 + s],
                device_id=(p ^ _PIPES[pi]["stages"][s][0],),
                device_id_type=pl.DeviceIdType.MESH,
            )

        def rs_add(pi, s, combo):
            keep = pi * half + sum(combo[j] * sizes[j] for j in range(s + 1))
            out_ref[pl.ds(keep, sizes[s]), :] = (
                out_ref[pl.ds(keep, sizes[s]), :]
                + stage_ref[pl.ds(pi * st_sz + st_offs[s], sizes[s]), :]
            )

        def ag_rdma(pi, s, combo, ag_i):
            off = pi * half + sum(combo[j] * sizes[j] for j in range(s + 1))
            return pltpu.make_async_remote_copy(
                src_ref=out_ref.at[pl.ds(off, sizes[s]), :],
                dst_ref=out_ref.at[pl.ds(off, sizes[s]), :],
                send_sem=send_sems.at[pi * 6 + 3 + ag_i],
                recv_sem=recv_sems.at[pi * 6 + 3 + ag_i],
                device_id=(p ^ _PIPES[pi]["stages"][s][0],),
                device_id_type=pl.DeviceIdType.MESH,
            )

        for pi in range(2):
            for combo in combos(1):
                @pl.when(cond_for(pi, combo))
                def _(pi=pi, combo=combo):
                    rs_rdma(pi, 0, combo).start()

        out_ref[...] = p_ref[...]

        for pi in range(2):
            for combo in combos(1):
                @pl.when(cond_for(pi, combo))
                def _(pi=pi, combo=combo):
                    rs_rdma(pi, 0, combo).wait()
                    rs_add(pi, 0, combo)

        for s in (1, 2):
            for pi in range(2):
                for combo in combos(s + 1):
                    @pl.when(cond_for(pi, combo))
                    def _(pi=pi, s=s, combo=combo):
                        rs_rdma(pi, s, combo).start()
            for pi in range(2):
                for combo in combos(s + 1):
                    @pl.when(cond_for(pi, combo))
                    def _(pi=pi, s=s, combo=combo):
                        rs_rdma(pi, s, combo).wait()
                        rs_add(pi, s, combo)

        for ag_i, s in enumerate((2, 1, 0)):
            for pi in range(2):
                for combo in combos(s + 1):
                    @pl.when(cond_for(pi, combo))
                    def _(pi=pi, s=s, combo=combo, ag_i=ag_i):
                        ag_rdma(pi, s, combo, ag_i).start()
            for pi in range(2):
                for combo in combos(s + 1):
                    @pl.when(cond_for(pi, combo))
                    def _(pi=pi, s=s, combo=combo, ag_i=ag_i):
                        ag_rdma(pi, s, combo, ag_i).wait()

    return pl.pallas_call(
        body,
        out_shape=jax.ShapeDtypeStruct((n, d), jnp.bfloat16),
        in_specs=[pl.BlockSpec(memory_space=pltpu.VMEM)],
        out_specs=pl.BlockSpec(memory_space=pltpu.VMEM),
        scratch_shapes=[
            pltpu.VMEM((2 * st_sz, d), jnp.bfloat16),
            pltpu.SemaphoreType.DMA((12,)),
            pltpu.SemaphoreType.DMA((12,)),
        ],
        compiler_params=pltpu.CompilerParams(collective_id=0),
    )(partial)
